# baseline (speedup 1.0000x reference)
import jax
import jax.numpy as jnp
from jax import lax
from jax.experimental import pallas as pl
from jax.experimental.pallas import tpu as pltpu
from jax.experimental.pallas import tpu_sc as plsc

NC, NS = 2, 16
NW = NC * NS
B, L, D = 4096, 200, 64
V = 1000000
EN = 5
PD = 8
IPG = 200
GPW = B // NW
BPW = B // NW
NG = 128 // PD
VG = V // NG
PC = 4096
DG = [(VG * g) % PC for g in range(NG)]
OG = [VG * g - DG[g] for g in range(NG)]
NB = -(-(VG + max(DG)) // PC)
VP = NB * PC


def _project(tableT, W16s):
    ncb = -(-V // PC)
    in_specs = [pl.BlockSpec((PD, D), lambda i: (0, 0))] + [
        pl.BlockSpec(
            (D, PC),
            lambda i, g=g: (0, jnp.minimum(OG[g] // PC + i, ncb - 1)),
        )
        for g in range(NG)
    ]

    def body(w_ref, *refs):
        t_refs, o_ref = refs[:-1], refs[-1]
        xs = [
            lax.dot_general(
                w_ref[...], t_refs[g][...],
                (((1,), (0,)), ((), ())),
                preferred_element_type=jnp.float32,
            )
            for g in range(NG)
        ]
        o_ref[...] = lax.transpose(jnp.concatenate(xs, axis=0), (1, 0))

    return pl.pallas_call(
        body,
        grid=(NB,),
        in_specs=in_specs,
        out_specs=pl.BlockSpec((PC, 128), lambda i: (i, 0)),
        out_shape=jax.ShapeDtypeStruct((VP, 128), jnp.float32),
        compiler_params=pltpu.CompilerParams(fuse_transposed_lhs_in_matmul=True),
    )(W16s, *([tableT] * NG))


def _gsum(rows_v, j, h2, c8):
    rv = rows_v.at[j]
    vals = [
        plsc.load_gather(rv, [h2 + 2 * k, c8]) for k in range(IPG // 2)
    ]
    while len(vals) > 1:
        vals = [
            vals[i] + vals[i + 1] if i + 1 < len(vals) else vals[i]
            for i in range(0, len(vals), 2)
        ]
    return vals[0]


NCHUNK = BPW * IPG // 16
NBUF = 8


def _sc_body(txt_hbm, p_hbm, out_hbm, raw_v, idx_v, rows_v, out_v, *sems):
    w = lax.axis_index("s") * NC + lax.axis_index("c")
    pltpu.sync_copy(txt_hbm.at[pl.ds(w * BPW * IPG, BPW * IPG)], raw_v)
    i16 = lax.iota(jnp.int32, 16)
    h2 = i16 >> 3
    c8 = i16 & 7

    def remap(c):
        t = raw_v[pl.ds(c * 16, 16)]
        g = ((t.astype(jnp.float32) + 0.5) * (1.0 / VG)).astype(jnp.int32)
        gm = g * VG
        idx_v[pl.ds(c * 16, 16)] = ((t - gm + (gm & (PC - 1))) << 4) + g

    def remap_many(c0, n):
        def rbody(k, _):
            remap(jnp.minimum(c0 + k, NCHUNK - 1))
            return 0
        lax.fori_loop(0, n, rbody, 0)

    remap_many(0, 104)

    def islice(g):
        return idx_v.at[pl.ds(g * IPG, IPG)]

    for j in range(NBUF - 1):
        pltpu.async_copy(p_hbm.at[islice(j)], rows_v.at[j], sems[j])

    def row_oct(rr, _):
        g0 = rr * NBUF
        for j in range(NBUF):
            remap_many(104 + (g0 + j) * 13, 13)
            nxt = jnp.minimum(g0 + j + NBUF - 1, GPW - 1)
            pltpu.async_copy(
                p_hbm.at[islice(nxt)],
                rows_v.at[(j + NBUF - 1) % NBUF],
                sems[(j + NBUF - 1) % NBUF],
            )
            pltpu.make_async_copy(
                p_hbm.at[islice(g0 + j)], rows_v.at[j], sems[j]
            ).wait()
            out_v[g0 + j, :] = _gsum(rows_v, j, h2, c8)
        return 0

    lax.fori_loop(0, BPW // NBUF, row_oct, 0)
    for j in range(NBUF - 1):
        pltpu.make_async_copy(
            p_hbm.at[islice(GPW - 1)], rows_v.at[j], sems[j]
        ).wait()
    pltpu.sync_copy(out_v, out_hbm.at[pl.ds(w * BPW, BPW)])


def _sc_pool(arr, p_lin):
    mesh = plsc.VectorSubcoreMesh(core_axis_name="c", subcore_axis_name="s")
    return pl.kernel(
        _sc_body,
        out_type=jax.ShapeDtypeStruct((B, 16), jnp.float32),
        mesh=mesh,
        scratch_types=[
            pltpu.VMEM((BPW * IPG,), jnp.int32),
            pltpu.VMEM((BPW * IPG,), jnp.int32),
            pltpu.VMEM((NBUF, IPG, PD), jnp.float32),
            pltpu.VMEM((BPW, 16), jnp.float32),
        ] + [pltpu.SemaphoreType.DMA] * NBUF,
        compiler_params=pltpu.CompilerParams(
            use_tc_tiling_on_sc=False, needs_layout_passes=False
        ),
    )(arr, p_lin)


def _tail_body(s_ref, b_ref, o_ref):
    s = s_ref[...]
    logits = s[:, :EN] + s[:, PD:PD + EN] + b_ref[...]
    m = jnp.max(logits, axis=1, keepdims=True)
    sh = logits - m
    o_ref[...] = sh - jnp.log(jnp.sum(jnp.exp(sh), axis=1, keepdims=True))


def _tail(sums, b):
    return pl.pallas_call(
        _tail_body,
        out_shape=jax.ShapeDtypeStruct((B, EN), jnp.float32),
    )(sums, b.reshape(1, EN))


def kernel(text, table, W, b):
    W16s = jnp.zeros((PD, D), jnp.float32).at[:EN].set(W) * (1.0 / L)
    p_pack = _project(table.T, W16s)
    p_lin = p_pack.reshape(NG * VP, PD)
    sums = _sc_pool(text.astype(jnp.int32).reshape(-1), p_lin)
    return _tail(sums, b)


# speedup vs baseline: 1.1087x; 1.1087x over previous
import jax
import jax.numpy as jnp
from jax import lax
from jax.experimental import pallas as pl
from jax.experimental.pallas import tpu as pltpu
from jax.experimental.pallas import tpu_sc as plsc

NC, NS = 2, 16
NW = NC * NS
B, L, D = 4096, 200, 64
V = 1000000
EN = 5
PD = 8
IPG = 200
GPW = B // NW
BPW = B // NW
NG = 128 // PD
VG = V // NG
PC = 4096
DG = [(VG * g) % PC for g in range(NG)]
OG = [VG * g - DG[g] for g in range(NG)]
NB = -(-(VG + max(DG)) // PC)
VP = NB * PC


def _project(tableT, W16s):
    ncb = -(-V // PC)
    in_specs = [pl.BlockSpec((PD, D), lambda i: (0, 0))] + [
        pl.BlockSpec(
            (D, PC),
            lambda i, g=g: (0, jnp.minimum(OG[g] // PC + i, ncb - 1)),
        )
        for g in range(NG)
    ]

    def body(w_ref, *refs):
        t_refs, o_ref = refs[:-1], refs[-1]
        xs = [
            lax.dot_general(
                w_ref[...], t_refs[g][...],
                (((1,), (0,)), ((), ())),
                preferred_element_type=jnp.float32,
            )
            for g in range(NG)
        ]
        o_ref[...] = lax.transpose(jnp.concatenate(xs, axis=0), (1, 0))

    return pl.pallas_call(
        body,
        grid=(NB,),
        in_specs=in_specs,
        out_specs=pl.BlockSpec((PC, 128), lambda i: (i, 0)),
        out_shape=jax.ShapeDtypeStruct((VP, 128), jnp.float32),
        compiler_params=pltpu.CompilerParams(fuse_transposed_lhs_in_matmul=True),
    )(W16s, *([tableT] * NG))


def _gsum(rows_v, j, h2, c8):
    rv = rows_v.at[j]
    vals = [
        plsc.load_gather(rv, [h2 + 2 * k, c8]) for k in range(IPG // 2)
    ]
    while len(vals) > 1:
        vals = [
            vals[i] + vals[i + 1] if i + 1 < len(vals) else vals[i]
            for i in range(0, len(vals), 2)
        ]
    return vals[0]


NCHUNK = BPW * IPG // 16
NBUF = 4


def _sc_body(txt_hbm, p_hbm, out_hbm, raw_v, idx_v, rows_v, out_v, *sems):
    w = lax.axis_index("s") * NC + lax.axis_index("c")
    pltpu.sync_copy(txt_hbm.at[pl.ds(w * BPW * IPG, BPW * IPG)], raw_v)
    i16 = lax.iota(jnp.int32, 16)
    h2 = i16 >> 3
    c8 = i16 & 7

    def remap(c):
        t = raw_v[pl.ds(c * 16, 16)]
        g = ((t.astype(jnp.float32) + 0.5) * (1.0 / VG)).astype(jnp.int32)
        gm = g * VG
        idx_v[pl.ds(c * 16, 16)] = ((t - gm + (gm & (PC - 1))) << 4) + g

    def remap_many(c0, n):
        def rbody(k, _):
            remap(jnp.minimum(c0 + k, NCHUNK - 1))
            return 0
        lax.fori_loop(0, n, rbody, 0)

    remap_many(0, 104)

    def islice(g):
        return idx_v.at[pl.ds(g * IPG, IPG)]

    for j in range(NBUF - 1):
        pltpu.async_copy(p_hbm.at[islice(j)], rows_v.at[j], sems[j])

    def row_oct(rr, _):
        g0 = rr * NBUF
        for j in range(NBUF):
            remap_many(104 + (g0 + j) * 13, 13)
            nxt = jnp.minimum(g0 + j + NBUF - 1, GPW - 1)
            pltpu.async_copy(
                p_hbm.at[islice(nxt)],
                rows_v.at[(j + NBUF - 1) % NBUF],
                sems[(j + NBUF - 1) % NBUF],
            )
            pltpu.make_async_copy(
                p_hbm.at[islice(g0 + j)], rows_v.at[j], sems[j]
            ).wait()
            out_v[g0 + j, :] = _gsum(rows_v, j, h2, c8)
        return 0

    lax.fori_loop(0, BPW // NBUF, row_oct, 0)
    for j in range(NBUF - 1):
        pltpu.make_async_copy(
            p_hbm.at[islice(GPW - 1)], rows_v.at[j], sems[j]
        ).wait()
    pltpu.sync_copy(out_v, out_hbm.at[pl.ds(w * BPW, BPW)])


def _sc_pool(arr, p_lin):
    mesh = plsc.VectorSubcoreMesh(core_axis_name="c", subcore_axis_name="s")
    return pl.kernel(
        _sc_body,
        out_type=jax.ShapeDtypeStruct((B, 16), jnp.float32),
        mesh=mesh,
        scratch_types=[
            pltpu.VMEM((BPW * IPG,), jnp.int32),
            pltpu.VMEM((BPW * IPG,), jnp.int32),
            pltpu.VMEM((NBUF, IPG, PD), jnp.float32),
            pltpu.VMEM((BPW, 16), jnp.float32),
        ] + [pltpu.SemaphoreType.DMA] * NBUF,
        compiler_params=pltpu.CompilerParams(
            use_tc_tiling_on_sc=False, needs_layout_passes=False
        ),
    )(arr, p_lin)


def _tail_body(s_ref, b_ref, o_ref):
    s = s_ref[...]
    logits = s[:, :EN] + s[:, PD:PD + EN] + b_ref[...]
    m = jnp.max(logits, axis=1, keepdims=True)
    sh = logits - m
    o_ref[...] = sh - jnp.log(jnp.sum(jnp.exp(sh), axis=1, keepdims=True))


def _tail(sums, b):
    return pl.pallas_call(
        _tail_body,
        out_shape=jax.ShapeDtypeStruct((B, EN), jnp.float32),
    )(sums, b.reshape(1, EN))


def kernel(text, table, W, b):
    W16s = jnp.zeros((PD, D), jnp.float32).at[:EN].set(W) * (1.0 / L)
    p_pack = _project(table.T, W16s)
    p_lin = p_pack.reshape(NG * VP, PD)
    sums = _sc_pool(text.astype(jnp.int32).reshape(-1), p_lin)
    return _tail(sums, b)


# speedup vs baseline: 1.1118x; 1.0028x over previous
import jax
import jax.numpy as jnp
from jax import lax
from jax.experimental import pallas as pl
from jax.experimental.pallas import tpu as pltpu
from jax.experimental.pallas import tpu_sc as plsc

NC, NS = 2, 16
NW = NC * NS
B, L, D = 4096, 200, 64
V = 1000000
EN = 5
PD = 8
IPG = 200
GPW = B // NW
BPW = B // NW
NG = 128 // PD
VG = V // NG
PC = 4096
DG = [(VG * g) % PC for g in range(NG)]
OG = [VG * g - DG[g] for g in range(NG)]
NB = -(-(VG + max(DG)) // PC)
VP = NB * PC


def _project(tableT, W16s):
    ncb = -(-V // PC)
    in_specs = [pl.BlockSpec((PD, D), lambda i: (0, 0))] + [
        pl.BlockSpec(
            (D, PC),
            lambda i, g=g: (0, jnp.minimum(OG[g] // PC + i, ncb - 1)),
        )
        for g in range(NG)
    ]

    def body(w_ref, *refs):
        t_refs, o_ref = refs[:-1], refs[-1]
        xs = [
            lax.dot_general(
                w_ref[...], t_refs[g][...],
                (((1,), (0,)), ((), ())),
                preferred_element_type=jnp.float32,
            )
            for g in range(NG)
        ]
        o_ref[...] = lax.transpose(jnp.concatenate(xs, axis=0), (1, 0))

    return pl.pallas_call(
        body,
        grid=(NB,),
        in_specs=in_specs,
        out_specs=pl.BlockSpec((PC, 128), lambda i: (i, 0)),
        out_shape=jax.ShapeDtypeStruct((VP, 128), jnp.float32),
        compiler_params=pltpu.CompilerParams(fuse_transposed_lhs_in_matmul=True),
    )(W16s, *([tableT] * NG))


def _gsum(rows_v, j, h2, c8):
    rv = rows_v.at[j]
    vals = [
        plsc.load_gather(rv, [h2 + 2 * k, c8]) for k in range(IPG // 2)
    ]
    while len(vals) > 1:
        vals = [
            vals[i] + vals[i + 1] if i + 1 < len(vals) else vals[i]
            for i in range(0, len(vals), 2)
        ]
    return vals[0]


NCHUNK = BPW * IPG // 16
NBUF = 4


def _logf(s):
    bits = plsc.bitcast(s, jnp.int32)
    e2 = ((bits >> 23) - 127).astype(jnp.float32)
    m = plsc.bitcast((bits & 0x7FFFFF) | 0x3F800000, jnp.float32)
    q = (m - 1.0) / (m + 1.0)
    q2 = q * q
    p = 2.0 / 9.0
    for c in (2.0 / 7.0, 2.0 / 5.0, 2.0 / 3.0, 2.0):
        p = p * q2 + c
    return e2 * 0.6931471805599453 + p * q


def _sc_body(txt_hbm, p_hbm, b_hbm, out_hbm, raw_v, idx_v, rows_v, out_v,
             out5_v, b_v, *sems):
    w = lax.axis_index("s") * NC + lax.axis_index("c")
    pltpu.sync_copy(txt_hbm.at[pl.ds(w * BPW * IPG, BPW * IPG)], raw_v)
    pltpu.sync_copy(b_hbm, b_v)
    i16 = lax.iota(jnp.int32, 16)
    h2 = i16 >> 3
    c8 = i16 & 7

    def remap(c):
        t = raw_v[pl.ds(c * 16, 16)]
        g = ((t.astype(jnp.float32) + 0.5) * (1.0 / VG)).astype(jnp.int32)
        gm = g * VG
        idx_v[pl.ds(c * 16, 16)] = ((t - gm + (gm & (PC - 1))) << 4) + g

    def remap_many(c0, n):
        def rbody(k, _):
            remap(jnp.minimum(c0 + k, NCHUNK - 1))
            return 0
        lax.fori_loop(0, n, rbody, 0)

    remap_many(0, 104)

    def islice(g):
        return idx_v.at[pl.ds(g * IPG, IPG)]

    for j in range(NBUF - 1):
        pltpu.async_copy(p_hbm.at[islice(j)], rows_v.at[j], sems[j])

    def row_oct(rr, _):
        g0 = rr * NBUF
        for j in range(NBUF):
            remap_many(104 + (g0 + j) * 13, 13)
            nxt = jnp.minimum(g0 + j + NBUF - 1, GPW - 1)
            pltpu.async_copy(
                p_hbm.at[islice(nxt)],
                rows_v.at[(j + NBUF - 1) % NBUF],
                sems[(j + NBUF - 1) % NBUF],
            )
            pltpu.make_async_copy(
                p_hbm.at[islice(g0 + j)], rows_v.at[j], sems[j]
            ).wait()
            out_v[g0 + j, :] = _gsum(rows_v, j, h2, c8)
        return 0

    lax.fori_loop(0, BPW // NBUF, row_oct, 0)
    for j in range(NBUF - 1):
        pltpu.make_async_copy(
            p_hbm.at[islice(GPW - 1)], rows_v.at[j], sems[j]
        ).wait()

    bv = b_v[:]
    lane_lt5 = i16 < EN
    e_idx = jnp.minimum(i16, EN - 1)
    perm8 = (i16 + 8) & 15

    def softmax_row(r, _):
        x = out_v[r, :]
        y = x + lax.gather(
            x, perm8[:, None],
            lax.GatherDimensionNumbers((), (0,), (0,)), (1,),
            mode=lax.GatherScatterMode.PROMISE_IN_BOUNDS,
        )
        z = jnp.where(lane_lt5, y + bv, -1e30)
        m = jnp.max(z, axis=0)
        zs = z - m
        s = jnp.sum(jnp.where(lane_lt5, jnp.exp(zs), 0.0), axis=0)
        res = zs - _logf(jnp.full((16,), s))
        plsc.store_scatter(
            out5_v, [e_idx, jnp.full((16,), r, jnp.int32)], res,
            mask=lane_lt5,
        )
        return 0

    lax.fori_loop(0, BPW, softmax_row, 0)
    pltpu.sync_copy(out5_v, out_hbm.at[:, pl.ds(w * BPW, BPW)])


def _sc_pool(arr, p_lin, b16):
    mesh = plsc.VectorSubcoreMesh(core_axis_name="c", subcore_axis_name="s")
    return pl.kernel(
        _sc_body,
        out_type=jax.ShapeDtypeStruct((EN, B), jnp.float32),
        mesh=mesh,
        scratch_types=[
            pltpu.VMEM((BPW * IPG,), jnp.int32),
            pltpu.VMEM((BPW * IPG,), jnp.int32),
            pltpu.VMEM((NBUF, IPG, PD), jnp.float32),
            pltpu.VMEM((BPW, 16), jnp.float32),
            pltpu.VMEM((EN, BPW), jnp.float32),
            pltpu.VMEM((16,), jnp.float32),
        ] + [pltpu.SemaphoreType.DMA] * NBUF,
        compiler_params=pltpu.CompilerParams(
            use_tc_tiling_on_sc=False, needs_layout_passes=False
        ),
    )(arr, p_lin, b16)


def kernel(text, table, W, b):
    W16s = jnp.zeros((PD, D), jnp.float32).at[:EN].set(W) * (1.0 / L)
    p_pack = _project(table.T, W16s)
    p_lin = p_pack.reshape(NG * VP, PD)
    b16 = jnp.zeros((16,), jnp.float32).at[:EN].set(b)
    out5 = _sc_pool(text.astype(jnp.int32).reshape(-1), p_lin, b16)
    return out5.T


# speedup vs baseline: 1.1514x; 1.0357x over previous
import jax
import jax.numpy as jnp
from jax import lax
from jax.experimental import pallas as pl
from jax.experimental.pallas import tpu as pltpu
from jax.experimental.pallas import tpu_sc as plsc

NC, NS = 2, 16
NW = NC * NS
B, L, D = 4096, 200, 64
V = 1000000
EN = 5
PD = 8
IPG = 200
GPW = B // NW
BPW = B // NW
NG = 128 // PD
VG = V // NG
PC = 2048
DG = [(VG * g) % PC for g in range(NG)]
OG = [VG * g - DG[g] for g in range(NG)]
NB = -(-(VG + max(DG)) // PC)
VP = NB * PC


def _project(tableT, W16s):
    ncb = -(-V // PC)
    in_specs = [pl.BlockSpec((PD, D), lambda i: (0, 0))] + [
        pl.BlockSpec(
            (D, PC),
            lambda i, g=g: (0, jnp.minimum(OG[g] // PC + i, ncb - 1)),
        )
        for g in range(NG)
    ]

    def body(w_ref, *refs):
        t_refs, o_ref = refs[:-1], refs[-1]
        xs = [
            lax.dot_general(
                w_ref[...], t_refs[g][...],
                (((1,), (0,)), ((), ())),
                preferred_element_type=jnp.float32,
            )
            for g in range(NG)
        ]
        o_ref[...] = lax.transpose(jnp.concatenate(xs, axis=0), (1, 0))

    return pl.pallas_call(
        body,
        grid=(NB,),
        in_specs=in_specs,
        out_specs=pl.BlockSpec((PC, 128), lambda i: (i, 0)),
        out_shape=jax.ShapeDtypeStruct((VP, 128), jnp.float32),
        compiler_params=pltpu.CompilerParams(fuse_transposed_lhs_in_matmul=True),
    )(W16s, *([tableT] * NG))


def _gsum(rows_v, j, h2, c8):
    rv = rows_v.at[j]
    vals = [
        plsc.load_gather(rv, [h2 + 2 * k, c8]) for k in range(IPG // 2)
    ]
    while len(vals) > 1:
        vals = [
            vals[i] + vals[i + 1] if i + 1 < len(vals) else vals[i]
            for i in range(0, len(vals), 2)
        ]
    return vals[0]


NCHUNK = BPW * IPG // 16
NBUF = 4


def _logf(s):
    bits = plsc.bitcast(s, jnp.int32)
    e2 = ((bits >> 23) - 127).astype(jnp.float32)
    m = plsc.bitcast((bits & 0x7FFFFF) | 0x3F800000, jnp.float32)
    q = (m - 1.0) / (m + 1.0)
    q2 = q * q
    p = 2.0 / 9.0
    for c in (2.0 / 7.0, 2.0 / 5.0, 2.0 / 3.0, 2.0):
        p = p * q2 + c
    return e2 * 0.6931471805599453 + p * q


def _sc_body(txt_hbm, p_hbm, b_hbm, out_hbm, raw_v, idx_v, rows_v, out_v,
             out5_v, b_v, *sems):
    w = lax.axis_index("s") * NC + lax.axis_index("c")
    pltpu.sync_copy(txt_hbm.at[pl.ds(w * BPW * IPG, BPW * IPG)], raw_v)
    pltpu.sync_copy(b_hbm, b_v)
    i16 = lax.iota(jnp.int32, 16)
    h2 = i16 >> 3
    c8 = i16 & 7

    def remap(c):
        t = raw_v[pl.ds(c * 16, 16)]
        g = ((t.astype(jnp.float32) + 0.5) * (1.0 / VG)).astype(jnp.int32)
        gm = g * VG
        idx_v[pl.ds(c * 16, 16)] = ((t - gm + (gm & (PC - 1))) << 4) + g

    def remap_many(c0, n):
        def rbody(k, _):
            remap(jnp.minimum(c0 + k, NCHUNK - 1))
            return 0
        lax.fori_loop(0, n, rbody, 0)

    remap_many(0, 104)

    def islice(g):
        return idx_v.at[pl.ds(g * IPG, IPG)]

    for j in range(NBUF - 1):
        pltpu.async_copy(p_hbm.at[islice(j)], rows_v.at[j], sems[j])

    def row_oct(rr, _):
        g0 = rr * NBUF
        for j in range(NBUF):
            remap_many(104 + (g0 + j) * 13, 13)
            nxt = jnp.minimum(g0 + j + NBUF - 1, GPW - 1)
            pltpu.async_copy(
                p_hbm.at[islice(nxt)],
                rows_v.at[(j + NBUF - 1) % NBUF],
                sems[(j + NBUF - 1) % NBUF],
            )
            pltpu.make_async_copy(
                p_hbm.at[islice(g0 + j)], rows_v.at[j], sems[j]
            ).wait()
            out_v[g0 + j, :] = _gsum(rows_v, j, h2, c8)
        return 0

    lax.fori_loop(0, BPW // NBUF, row_oct, 0)
    for j in range(NBUF - 1):
        pltpu.make_async_copy(
            p_hbm.at[islice(GPW - 1)], rows_v.at[j], sems[j]
        ).wait()

    bv = b_v[:]
    lane_lt5 = i16 < EN
    e_idx = jnp.minimum(i16, EN - 1)
    perm8 = (i16 + 8) & 15

    def softmax_row(r, _):
        x = out_v[r, :]
        y = x + lax.gather(
            x, perm8[:, None],
            lax.GatherDimensionNumbers((), (0,), (0,)), (1,),
            mode=lax.GatherScatterMode.PROMISE_IN_BOUNDS,
        )
        z = jnp.where(lane_lt5, y + bv, -1e30)
        m = jnp.max(z, axis=0)
        zs = z - m
        s = jnp.sum(jnp.where(lane_lt5, jnp.exp(zs), 0.0), axis=0)
        res = zs - _logf(jnp.full((16,), s))
        plsc.store_scatter(
            out5_v, [e_idx, jnp.full((16,), r, jnp.int32)], res,
            mask=lane_lt5,
        )
        return 0

    lax.fori_loop(0, BPW, softmax_row, 0)
    pltpu.sync_copy(out5_v, out_hbm.at[:, pl.ds(w * BPW, BPW)])


def _sc_pool(arr, p_lin, b16):
    mesh = plsc.VectorSubcoreMesh(core_axis_name="c", subcore_axis_name="s")
    return pl.kernel(
        _sc_body,
        out_type=jax.ShapeDtypeStruct((EN, B), jnp.float32),
        mesh=mesh,
        scratch_types=[
            pltpu.VMEM((BPW * IPG,), jnp.int32),
            pltpu.VMEM((BPW * IPG,), jnp.int32),
            pltpu.VMEM((NBUF, IPG, PD), jnp.float32),
            pltpu.VMEM((BPW, 16), jnp.float32),
            pltpu.VMEM((EN, BPW), jnp.float32),
            pltpu.VMEM((16,), jnp.float32),
        ] + [pltpu.SemaphoreType.DMA] * NBUF,
        compiler_params=pltpu.CompilerParams(
            use_tc_tiling_on_sc=False, needs_layout_passes=False
        ),
    )(arr, p_lin, b16)


def kernel(text, table, W, b):
    W16s = jnp.zeros((PD, D), jnp.float32).at[:EN].set(W) * (1.0 / L)
    p_pack = _project(table.T, W16s)
    p_lin = p_pack.reshape(NG * VP, PD)
    b16 = jnp.zeros((16,), jnp.float32).at[:EN].set(b)
    out5 = _sc_pool(text.astype(jnp.int32).reshape(-1), p_lin, b16)
    return out5.T
